# baseline (speedup 1.0000x reference)
import functools

import jax
import jax.numpy as jnp
from jax import lax
from jax.experimental import pallas as pl
from jax.experimental.pallas import tpu as pltpu
from jax.experimental.pallas import tpu_sc as plsc

N = 10000
E = 320000
F_IN = 128
HID = 16
G = 256

NC = 2
NS = 16
NW = NC * NS
EPW = E // NW
CHUNK = 80
NCHUNK = EPW // CHUNK
NPAD = 10240
ROWS_PER_SUB = NPAD // NS


def _lrelu(v):
    return jnp.where(v >= 0, v, 0.01 * v)


def _mm_body(x_ref, w_ref, o_ref):
    o_ref[...] = jnp.dot(x_ref[...], w_ref[...],
                         preferred_element_type=jnp.float32)


def _matmul(x, W1):
    return pl.pallas_call(
        _mm_body,
        out_shape=jax.ShapeDtypeStruct((N, HID), jnp.float32),
    )(x, W1)


NBUF = 5
NOUT = NCHUNK // NBUF


def _edge_body(h_hbm, ei_hbm, w_hbm, out_hbm,
               acc, zbuf, src_all, dst_all, w_all, gbuf, sbuf,
               sem_in, sem_g, sem_s):
    cid = lax.axis_index("c")
    sid = lax.axis_index("s")
    wid = cid * NS + sid

    cp_s = pltpu.async_copy(ei_hbm.at[0, wid], src_all, sem_in)
    cp_d = pltpu.async_copy(ei_hbm.at[1, wid], dst_all, sem_in)
    cp_w = pltpu.async_copy(w_hbm.at[wid], w_all, sem_in)

    def _zero_row(i, _):
        zbuf[i, :] = jnp.zeros((16,), jnp.float32)
        return _
    lax.fori_loop(0, ROWS_PER_SUB, _zero_row, None)
    pltpu.sync_copy(zbuf, acc.at[pl.ds(sid * ROWS_PER_SUB, ROWS_PER_SUB)])
    cp_s.wait()
    cp_d.wait()
    cp_w.wait()

    for b in range(NBUF):
        pltpu.async_copy(h_hbm.at[src_all.at[b]], gbuf.at[b], sem_g.at[b])
    plsc.subcore_barrier()

    def _outer(jj, _):
        for b in range(NBUF):
            i = jj * NBUF + b
            pltpu.make_async_copy(h_hbm.at[pl.ds(0, CHUNK)], gbuf.at[b],
                                  sem_g.at[b]).wait()

            @pl.when(jj > 0)
            def _drain():
                pltpu.make_async_copy(sbuf.at[b], acc.at[pl.ds(0, CHUNK)],
                                      sem_s.at[b]).wait()

            for e16 in range(CHUNK // 16):
                wv = w_all[i, pl.ds(e16 * 16, 16)]
                for l in range(16):
                    e = e16 * 16 + l
                    sbuf[b, e, :] = gbuf[b, e, :] * wv[l]

            pltpu.async_copy(sbuf.at[b], acc.at[dst_all.at[i]],
                             sem_s.at[b], add=True)

            @pl.when(jj < NOUT - 1)
            def _refill():
                pltpu.async_copy(h_hbm.at[src_all.at[i + NBUF]], gbuf.at[b],
                                 sem_g.at[b])
        return _
    lax.fori_loop(0, NOUT, _outer, None)

    for b in range(NBUF):
        pltpu.make_async_copy(sbuf.at[b], acc.at[pl.ds(0, CHUNK)],
                              sem_s.at[b]).wait()
    plsc.subcore_barrier()
    r0 = sid * ROWS_PER_SUB
    pltpu.sync_copy(acc.at[pl.ds(r0, ROWS_PER_SUB)],
                    out_hbm.at[cid, pl.ds(r0, ROWS_PER_SUB)])


def _edge_agg(h, edge_index, w):
    mesh = plsc.VectorSubcoreMesh(core_axis_name="c", subcore_axis_name="s")
    k = pl.kernel(
        _edge_body,
        out_type=jax.ShapeDtypeStruct((NC, NPAD, HID), jnp.float32),
        mesh=mesh,
        scratch_types=[
            pltpu.VMEM_SHARED((NPAD, HID), jnp.float32),
            pltpu.VMEM((ROWS_PER_SUB, HID), jnp.float32),
            pltpu.VMEM((NCHUNK, CHUNK), jnp.int32),
            pltpu.VMEM((NCHUNK, CHUNK), jnp.int32),
            pltpu.VMEM((NCHUNK, CHUNK), jnp.float32),
            pltpu.VMEM((NBUF, CHUNK, HID), jnp.float32),
            pltpu.VMEM((NBUF, CHUNK, HID), jnp.float32),
            pltpu.SemaphoreType.DMA,
            pltpu.SemaphoreType.DMA((NBUF,)),
            pltpu.SemaphoreType.DMA((NBUF,)),
        ],
        compiler_params=pltpu.CompilerParams(use_tc_tiling_on_sc=False),
    )
    return k(h, edge_index.reshape(2, NW, NCHUNK, CHUNK),
             w.reshape(NW, NCHUNK, CHUNK))


def _epi_body(p_ref, batch_ref, b1_ref, gam_ref, bet_ref,
              w1_ref, c1_ref, w2_ref, c2_ref, w3_ref, c3_ref, o_ref):
    agg = p_ref[0, :N, :] + p_ref[1, :N, :] + b1_ref[...]
    h = _lrelu(agg)
    mean = jnp.mean(h, axis=0, keepdims=True)
    var = jnp.mean((h - mean) ** 2, axis=0, keepdims=True)
    hn = (h - mean) * lax.rsqrt(var + 1e-5) * gam_ref[...] + bet_ref[...]
    h2 = _lrelu(hn)
    gids = lax.broadcasted_iota(jnp.int32, (G, N), 0)
    onehot = (gids == batch_ref[...]).astype(jnp.float32)
    h2_hi = h2.astype(jnp.bfloat16).astype(jnp.float32)
    rem = h2 - h2_hi
    h2_mid = rem.astype(jnp.bfloat16).astype(jnp.float32)
    h2_lo = rem - h2_mid
    pool = (jnp.dot(onehot, h2_hi, preferred_element_type=jnp.float32)
            + jnp.dot(onehot, h2_mid, preferred_element_type=jnp.float32)
            + jnp.dot(onehot, h2_lo, preferred_element_type=jnp.float32))

    o1 = _lrelu(jnp.dot(pool, w1_ref[...],
                        preferred_element_type=jnp.float32) + c1_ref[...])
    o2 = _lrelu(jnp.dot(o1, w2_ref[...],
                        preferred_element_type=jnp.float32) + c2_ref[...])
    o_ref[...] = _lrelu(jnp.dot(o2, w3_ref[...],
                                preferred_element_type=jnp.float32) + c3_ref[...])


def _epilogue(partials, batch, b1, gamma, beta,
              fc1_w, fc1_b, fc2_w, fc2_b, fc3_w, fc3_b):
    return pl.pallas_call(
        _epi_body,
        out_shape=jax.ShapeDtypeStruct((G, 2), jnp.float32),
    )(partials, batch.reshape(1, N), b1.reshape(1, HID),
      gamma.reshape(1, HID), beta.reshape(1, HID),
      fc1_w, fc1_b.reshape(1, 8), fc2_w, fc2_b.reshape(1, 4),
      fc3_w, fc3_b.reshape(1, 2))


def kernel(x, edge_index, edge_weigth, batch, W1, b1, gamma, beta,
           fc1_w, fc1_b, fc2_w, fc2_b, fc3_w, fc3_b):
    h = _matmul(x, W1)
    partials = _edge_agg(h, edge_index, edge_weigth)
    return _epilogue(partials, batch, b1, gamma, beta,
                     fc1_w, fc1_b, fc2_w, fc2_b, fc3_w, fc3_b)


# speedup vs baseline: 1.0794x; 1.0794x over previous
import functools

import jax
import jax.numpy as jnp
from jax import lax
from jax.experimental import pallas as pl
from jax.experimental.pallas import tpu as pltpu
from jax.experimental.pallas import tpu_sc as plsc

N = 10000
E = 320000
F_IN = 128
HID = 16
G = 256

NC = 2
NS = 16
NW = NC * NS
EPW = E // NW
CHUNK = 80
NCHUNK = EPW // CHUNK
NPAD = 10240
ROWS_PER_SUB = NPAD // NS


def _lrelu(v):
    return jnp.where(v >= 0, v, 0.01 * v)


def _mm_body(x_ref, w_ref, o_ref):
    o_ref[...] = jnp.dot(x_ref[...], w_ref[...],
                         preferred_element_type=jnp.float32)


def _matmul(x, W1):
    return pl.pallas_call(
        _mm_body,
        out_shape=jax.ShapeDtypeStruct((N, HID), jnp.float32),
    )(x, W1)


NBUF = 5
NOUT = NCHUNK // NBUF


H_ROWS_PER_SUB = N // NS


def _edge_body(h_hbm, ei_hbm, w_hbm, out_hbm,
               h_sp, acc, zbuf, src_all, dst_all, w_all, gbuf, sbuf,
               sem_in, sem_g, sem_s):
    cid = lax.axis_index("c")
    sid = lax.axis_index("s")
    wid = cid * NS + sid
    base = wid * EPW

    cp_s = pltpu.async_copy(ei_hbm.at[0, pl.ds(base, EPW)], src_all, sem_in)
    cp_d = pltpu.async_copy(ei_hbm.at[1, pl.ds(base, EPW)], dst_all, sem_in)
    cp_w = pltpu.async_copy(w_hbm.at[pl.ds(base, EPW)], w_all, sem_in)

    hr0 = sid * H_ROWS_PER_SUB
    pltpu.sync_copy(h_hbm.at[pl.ds(hr0, H_ROWS_PER_SUB)],
                    h_sp.at[pl.ds(hr0, H_ROWS_PER_SUB)])

    def _zero_row(i, _):
        zbuf[i, :] = jnp.zeros((16,), jnp.float32)
        return _
    lax.fori_loop(0, ROWS_PER_SUB, _zero_row, None)
    pltpu.sync_copy(zbuf, acc.at[pl.ds(sid * ROWS_PER_SUB, ROWS_PER_SUB)])
    cp_s.wait()
    cp_d.wait()
    cp_w.wait()
    plsc.subcore_barrier()

    for b in range(NBUF):
        pltpu.async_copy(h_sp.at[src_all.at[pl.ds(b * CHUNK, CHUNK)]],
                         gbuf.at[b], sem_g.at[b])

    def _outer(jj, _):
        for b in range(NBUF):
            i = jj * NBUF + b
            pltpu.make_async_copy(h_hbm.at[pl.ds(0, CHUNK)], gbuf.at[b],
                                  sem_g.at[b]).wait()

            @pl.when(jj > 0)
            def _drain():
                pltpu.make_async_copy(sbuf.at[b], acc.at[pl.ds(0, CHUNK)],
                                      sem_s.at[b]).wait()

            for e16 in range(CHUNK // 16):
                wv = w_all[pl.ds(i * CHUNK + e16 * 16, 16)]
                for l in range(16):
                    e = e16 * 16 + l
                    sbuf[b, e, :] = gbuf[b, e, :] * wv[l]

            pltpu.async_copy(
                sbuf.at[b], acc.at[dst_all.at[pl.ds(i * CHUNK, CHUNK)]],
                sem_s.at[b], add=True)

            @pl.when(jj < NOUT - 1)
            def _refill():
                pltpu.async_copy(
                    h_sp.at[src_all.at[pl.ds((i + NBUF) * CHUNK, CHUNK)]],
                    gbuf.at[b], sem_g.at[b])
        return _
    lax.fori_loop(0, NOUT, _outer, None)

    for b in range(NBUF):
        pltpu.make_async_copy(sbuf.at[b], acc.at[pl.ds(0, CHUNK)],
                              sem_s.at[b]).wait()
    plsc.subcore_barrier()
    r0 = sid * ROWS_PER_SUB
    pltpu.sync_copy(acc.at[pl.ds(r0, ROWS_PER_SUB)],
                    out_hbm.at[cid, pl.ds(r0, ROWS_PER_SUB)])


def _edge_agg(h, edge_index, w):
    mesh = plsc.VectorSubcoreMesh(core_axis_name="c", subcore_axis_name="s")
    k = pl.kernel(
        _edge_body,
        out_type=jax.ShapeDtypeStruct((NC, NPAD, HID), jnp.float32),
        mesh=mesh,
        scratch_types=[
            pltpu.VMEM_SHARED((NPAD, HID), jnp.float32),
            pltpu.VMEM_SHARED((NPAD, HID), jnp.float32),
            pltpu.VMEM((ROWS_PER_SUB, HID), jnp.float32),
            pltpu.VMEM((EPW,), jnp.int32),
            pltpu.VMEM((EPW,), jnp.int32),
            pltpu.VMEM((EPW,), jnp.float32),
            pltpu.VMEM((NBUF, CHUNK, HID), jnp.float32),
            pltpu.VMEM((NBUF, CHUNK, HID), jnp.float32),
            pltpu.SemaphoreType.DMA,
            pltpu.SemaphoreType.DMA((NBUF,)),
            pltpu.SemaphoreType.DMA((NBUF,)),
        ],
        compiler_params=pltpu.CompilerParams(use_tc_tiling_on_sc=False),
    )
    return k(h, edge_index, w)


def _epi_body(p_ref, batch_ref, b1_ref, gam_ref, bet_ref,
              w1_ref, c1_ref, w2_ref, c2_ref, w3_ref, c3_ref, o_ref):
    agg = p_ref[0, :N, :] + p_ref[1, :N, :] + b1_ref[...]
    h = _lrelu(agg)
    mean = jnp.mean(h, axis=0, keepdims=True)
    var = jnp.mean((h - mean) ** 2, axis=0, keepdims=True)
    hn = (h - mean) * lax.rsqrt(var + 1e-5) * gam_ref[...] + bet_ref[...]
    h2 = _lrelu(hn)
    gids = lax.broadcasted_iota(jnp.int32, (G, N), 0)
    onehot = (gids == batch_ref[...]).astype(jnp.float32)
    h2_hi = h2.astype(jnp.bfloat16).astype(jnp.float32)
    rem = h2 - h2_hi
    h2_mid = rem.astype(jnp.bfloat16).astype(jnp.float32)
    h2_lo = rem - h2_mid
    pool = (jnp.dot(onehot, h2_hi, preferred_element_type=jnp.float32)
            + jnp.dot(onehot, h2_mid, preferred_element_type=jnp.float32)
            + jnp.dot(onehot, h2_lo, preferred_element_type=jnp.float32))

    o1 = _lrelu(jnp.dot(pool, w1_ref[...],
                        preferred_element_type=jnp.float32) + c1_ref[...])
    o2 = _lrelu(jnp.dot(o1, w2_ref[...],
                        preferred_element_type=jnp.float32) + c2_ref[...])
    o_ref[...] = _lrelu(jnp.dot(o2, w3_ref[...],
                                preferred_element_type=jnp.float32) + c3_ref[...])


def _epilogue(partials, batch, b1, gamma, beta,
              fc1_w, fc1_b, fc2_w, fc2_b, fc3_w, fc3_b):
    return pl.pallas_call(
        _epi_body,
        out_shape=jax.ShapeDtypeStruct((G, 2), jnp.float32),
    )(partials, batch.reshape(1, N), b1.reshape(1, HID),
      gamma.reshape(1, HID), beta.reshape(1, HID),
      fc1_w, fc1_b.reshape(1, 8), fc2_w, fc2_b.reshape(1, 4),
      fc3_w, fc3_b.reshape(1, 2))


def kernel(x, edge_index, edge_weigth, batch, W1, b1, gamma, beta,
           fc1_w, fc1_b, fc2_w, fc2_b, fc3_w, fc3_b):
    h = _matmul(x, W1)
    partials = _edge_agg(h, edge_index, edge_weigth)
    return _epilogue(partials, batch, b1, gamma, beta,
                     fc1_w, fc1_b, fc2_w, fc2_b, fc3_w, fc3_b)
